# baseline (speedup 1.0000x reference)
import functools

import jax
import jax.numpy as jnp
import numpy as np
from jax import lax
from jax.experimental import pallas as pl
from jax.experimental.pallas import tpu as pltpu
from jax.experimental.pallas import tpu_sc as plsc

N_GENES = 10000
D = 128
N_EDGES = 320000

NC = 2
NS = 16
L = 16

NPAD = 10240
EDGES_PER_TILE_DEG = N_EDGES // NS
DEG_CHUNK = 2000
AGG_CHUNK = 64
AGG_NCHUNK = 160
AGG_SEG = 32
AGG_NBUF = 5
EPAD = NC * NS * AGG_NCHUNK * AGG_CHUNK
TRASH_ROW = N_GENES + 16
NACC = 10112
ACC_PER_TILE = NACC // NS

_N_EPAD = EPAD - N_EDGES
_SRC_PAD = np.asarray((np.arange(_N_EPAD) * 131) % N_GENES,
                      np.int32).reshape(-1, AGG_CHUNK)
_DST_PAD = np.asarray(N_GENES + np.arange(_N_EPAD) % (NACC - N_GENES),
                      np.int32).reshape(-1, AGG_CHUNK)


def _newton_rsqrt(x):
    i = plsc.bitcast(x, jnp.int32)
    y = plsc.bitcast(jnp.int32(0x5F3759DF) - (i >> 1), jnp.float32)
    for _ in range(3):
        y = y * (1.5 - 0.5 * x * y * y)
    return y


DEG_R = 128
DEG_ROWS_PER_TILE = EPAD // AGG_CHUNK // NS


def _deg_call(e3):
    mesh = plsc.VectorSubcoreMesh(core_axis_name="c", subcore_axis_name="s")

    @functools.partial(
        pl.kernel,
        out_type=jax.ShapeDtypeStruct((DEG_R, 128), jnp.float32),
        mesh=mesh,
        scratch_types=[
            pltpu.VMEM((DEG_ROWS_PER_TILE, AGG_CHUNK), jnp.int32),
            pltpu.VMEM((DEG_R, 128), jnp.float32),
            pltpu.VMEM((8, 128), jnp.float32),
            pltpu.VMEM((DEG_R,), jnp.int32),
            pltpu.VMEM_SHARED((DEG_R, 128), jnp.float32),
        ],
        compiler_params=pltpu.CompilerParams(needs_layout_passes=False),
    )
    def call(e3_hbm, dinv_hbm, dstbuf, deg2d, tmp, idx128, deg_sh):
        cid = lax.axis_index("c")
        sid = lax.axis_index("s")

        @pl.when(cid == 0)
        def _():
            zeros16 = jnp.zeros((L,), jnp.float32)
            ones = zeros16 + 1.0

            for i in range(8):
                for j in range(8):
                    tmp[i, pl.ds(j * L, L)] = zeros16
            pltpu.sync_copy(tmp, deg_sh.at[pl.ds(sid * 8, 8)])

            def z(i, _):
                for j in range(8):
                    deg2d[i, pl.ds(j * L, L)] = zeros16
                return 0
            lax.fori_loop(0, DEG_R, z, 0)
            for i in range(8):
                idx128[pl.ds(i * L, L)] = lax.iota(jnp.int32, L) + (i * L)

            pltpu.sync_copy(
                e3_hbm.at[1, pl.ds(sid * DEG_ROWS_PER_TILE,
                                   DEG_ROWS_PER_TILE), :], dstbuf)

            def scat(r, _):
                for u in range(AGG_CHUNK // L):
                    idx = dstbuf[r, pl.ds(u * L, L)]
                    row = lax.shift_right_logical(idx, 7)
                    col = jnp.bitwise_and(idx, 127)
                    plsc.addupdate_scatter(deg2d, [row, col], ones)
                return 0
            lax.fori_loop(0, DEG_ROWS_PER_TILE, scat, 0)

            plsc.subcore_barrier()
            pltpu.sync_copy(deg2d, deg_sh.at[idx128], add=True)
            plsc.subcore_barrier()

            pltpu.sync_copy(deg_sh.at[pl.ds(sid * 8, 8)], tmp)
            for i in range(8):
                for j in range(8):
                    d = tmp[i, pl.ds(j * L, L)] + 1.0
                    tmp[i, pl.ds(j * L, L)] = _newton_rsqrt(d)
            pltpu.sync_copy(tmp, dinv_hbm.at[pl.ds(sid * 8, 8), :])

    return call(e3)


def _agg_call(hs, e3):
    mesh = plsc.VectorSubcoreMesh(core_axis_name="c", subcore_axis_name="s")

    @functools.partial(
        pl.kernel,
        out_type=[jax.ShapeDtypeStruct((NACC, D), jnp.float32),
                  jax.ShapeDtypeStruct((NACC, D), jnp.float32)],
        mesh=mesh,
        scratch_types=[
            pltpu.VMEM((AGG_SEG, AGG_CHUNK), jnp.int32),
            pltpu.VMEM((AGG_SEG, AGG_CHUNK), jnp.int32),
        ] + [pltpu.VMEM((AGG_CHUNK, D), jnp.float32)] * AGG_NBUF + [
            pltpu.VMEM_SHARED((NACC, D), jnp.float32),
        ] + [pltpu.SemaphoreType.DMA] * AGG_NBUF,
        compiler_params=pltpu.CompilerParams(needs_layout_passes=False),
    )
    def call(hs_hbm, e3_hbm, out0_hbm, out1_hbm, sidx, didx, rows0,
             rows1, rows2, rows3, rows4, agg_sh, gsem0, gsem1, gsem2, gsem3,
             gsem4):
        cid = lax.axis_index("c")
        sid = lax.axis_index("s")
        wid = cid * NS + sid
        rows = [rows0, rows1, rows2, rows3, rows4]
        gsems = [gsem0, gsem1, gsem2, gsem3, gsem4]
        nb = AGG_NBUF

        def zl(i, _):
            for j in range(D // L):
                rows0[i, pl.ds(j * L, L)] = jnp.zeros((L,), jnp.float32)
            return 0
        lax.fori_loop(0, AGG_CHUNK, zl, 0)
        sbase = sid * ACC_PER_TILE
        nz = ACC_PER_TILE // AGG_CHUNK
        for r in range(nz):
            pltpu.sync_copy(rows0, agg_sh.at[pl.ds(sbase + r * AGG_CHUNK,
                                                   AGG_CHUNK)])
        pltpu.sync_copy(rows0.at[pl.ds(0, ACC_PER_TILE - nz * AGG_CHUNK)],
                        agg_sh.at[pl.ds(sbase + nz * AGG_CHUNK,
                                        ACC_PER_TILE - nz * AGG_CHUNK)])
        plsc.subcore_barrier()

        dummy = hs_hbm.at[pl.ds(0, AGG_CHUNK)]
        rowbase = wid * AGG_NCHUNK

        for q in range(AGG_NCHUNK // AGG_SEG):
            qb = rowbase + q * AGG_SEG
            pltpu.sync_copy(e3_hbm.at[0, pl.ds(qb, AGG_SEG), :], sidx)
            pltpu.sync_copy(e3_hbm.at[1, pl.ds(qb, AGG_SEG), :], didx)

            for b in range(nb - 1):
                pltpu.async_copy(hs_hbm.at[sidx.at[b]], rows[b], gsems[b])

            nring = (AGG_SEG - (nb - 1) - 3) // nb

            def ring(g, _):
                k = nb * g
                for b in range(nb):
                    c = k + b
                    fb = (b + nb - 1) % nb
                    pltpu.make_async_copy(dummy, rows[b], gsems[b]).wait()
                    pltpu.async_copy(hs_hbm.at[sidx.at[c + nb - 1]], rows[fb],
                                     gsems[fb])
                    pltpu.sync_copy(rows[b], agg_sh.at[didx.at[c]], add=True)
                return 0
            lax.fori_loop(0, nring, ring, 0)

            for c in range(nring * nb, AGG_SEG):
                b = c % nb
                pltpu.make_async_copy(dummy, rows[b], gsems[b]).wait()
                if c + nb - 1 < AGG_SEG:
                    fb = (b + nb - 1) % nb
                    pltpu.async_copy(hs_hbm.at[sidx.at[c + nb - 1]], rows[fb],
                                     gsems[fb])
                pltpu.sync_copy(rows[b], agg_sh.at[didx.at[c]], add=True)

        plsc.subcore_barrier()

        @pl.when(cid == 0)
        def _():
            pltpu.sync_copy(agg_sh.at[pl.ds(sbase, ACC_PER_TILE)],
                            out0_hbm.at[pl.ds(sbase, ACC_PER_TILE)])

        @pl.when(cid == 1)
        def _():
            pltpu.sync_copy(agg_sh.at[pl.ds(sbase, ACC_PER_TILE)],
                            out1_hbm.at[pl.ds(sbase, ACC_PER_TILE)])

    return call(hs, e3)


_BLKP = 1024
_BLK = 1000
_GRID = 10


def _tc1_body(x_ref, dinv_ref, w_ref, o_ref):
    o_ref[...] = jnp.dot(x_ref[...] * dinv_ref[...], w_ref[...],
                         preferred_element_type=jnp.float32)


def _tc2_body(p0_ref, p1_ref, hs_ref, dinv_ref, b_ref, w_ref, o_ref):
    agg = (p0_ref[...] + p1_ref[...] + hs_ref[...]) * dinv_ref[...]
    x1 = jnp.maximum(agg + b_ref[...], 0.0)
    o_ref[...] = jnp.dot(x1 * dinv_ref[...], w_ref[...],
                         preferred_element_type=jnp.float32)


def _tc3_body(p0_ref, p1_ref, hs_ref, dinv_ref, b_ref, o_ref):
    o_ref[...] = ((p0_ref[...] + p1_ref[...] + hs_ref[...]) * dinv_ref[...]
                  + b_ref[...])


def _row_spec():
    return pl.BlockSpec((_BLK, D), lambda i: (i, 0))


def _rowp_spec():
    return pl.BlockSpec((_BLKP, D), lambda i: (i, 0))


def _dinv_spec():
    return pl.BlockSpec((_BLKP, 1), lambda i: (i, 0))


def _dinv3_spec():
    return pl.BlockSpec((_BLK, 1), lambda i: (i, 0))


def _full_spec():
    return pl.BlockSpec((D, D), lambda i: (0, 0))


def _bias_spec():
    return pl.BlockSpec((1, D), lambda i: (0, 0))


def _tc1(x, dinv_bc, w):
    return pl.pallas_call(
        _tc1_body,
        grid=(_GRID,),
        in_specs=[_rowp_spec(), _dinv_spec(), _full_spec()],
        out_specs=_rowp_spec(),
        out_shape=jax.ShapeDtypeStruct((NPAD, D), jnp.float32),
    )(x, dinv_bc, w)


def _tc2(p0, p1, hs, dinv_bc, b, w):
    return pl.pallas_call(
        _tc2_body,
        grid=(_GRID,),
        in_specs=[_rowp_spec(), _rowp_spec(), _rowp_spec(), _dinv_spec(),
                  _bias_spec(), _full_spec()],
        out_specs=_rowp_spec(),
        out_shape=jax.ShapeDtypeStruct((NPAD, D), jnp.float32),
    )(p0, p1, hs, dinv_bc, b, w)


def _tc3(p0, p1, hs, dinv_bc, b):
    return pl.pallas_call(
        _tc3_body,
        grid=(_GRID,),
        in_specs=[_row_spec(), _row_spec(), _row_spec(), _dinv3_spec(),
                  _bias_spec()],
        out_specs=_row_spec(),
        out_shape=jax.ShapeDtypeStruct((N_GENES, D), jnp.float32),
    )(p0, p1, hs, dinv_bc, b)


def kernel(gene_ind_vec, edge_index, gene_embedding, W1, b1, W2, b2):
    pad3 = jnp.asarray(np.stack([_SRC_PAD, _DST_PAD]))
    e3 = jnp.concatenate(
        [edge_index.reshape(2, -1, AGG_CHUNK), pad3], axis=1)

    dinv2d = _deg_call(e3)
    dinv_col = dinv2d.reshape(-1)[:NPAD, None]
    x_pad = jnp.concatenate(
        [gene_embedding, jnp.zeros((NPAD - N_GENES, D), jnp.float32)])

    hs1 = _tc1(x_pad, dinv_col, W1)
    p0, p1 = _agg_call(hs1, e3)
    hs2 = _tc2(p0, p1, hs1, dinv_col, b1.reshape(1, D), W2)
    q0, q1 = _agg_call(hs2, e3)
    out = _tc3(q0, q1, hs2, dinv_col, b2.reshape(1, D))
    return out


# speedup vs baseline: 1.0593x; 1.0593x over previous
import functools

import jax
import jax.numpy as jnp
import numpy as np
from jax import lax
from jax.experimental import pallas as pl
from jax.experimental.pallas import tpu as pltpu
from jax.experimental.pallas import tpu_sc as plsc

N_GENES = 10000
D = 128
N_EDGES = 320000

NC = 2
NS = 16
L = 16

NPAD = 10240
EDGES_PER_TILE_DEG = N_EDGES // NS
DEG_CHUNK = 2000
AGG_CHUNK = 80
AGG_NCHUNK = 128
AGG_SEG = 32
AGG_NBUF = 4
EPAD = NC * NS * AGG_NCHUNK * AGG_CHUNK
TRASH_ROW = N_GENES + 16
NACC = 10112
ACC_PER_TILE = NACC // NS

_N_EPAD = EPAD - N_EDGES
_SRC_PAD = np.asarray((np.arange(_N_EPAD) * 131) % N_GENES,
                      np.int32).reshape(-1, AGG_CHUNK)
_DST_PAD = np.asarray(N_GENES + np.arange(_N_EPAD) % (NACC - N_GENES),
                      np.int32).reshape(-1, AGG_CHUNK)


def _newton_rsqrt(x):
    i = plsc.bitcast(x, jnp.int32)
    y = plsc.bitcast(jnp.int32(0x5F3759DF) - (i >> 1), jnp.float32)
    for _ in range(3):
        y = y * (1.5 - 0.5 * x * y * y)
    return y


DEG_R = 128
DEG_ROWS_PER_TILE = EPAD // AGG_CHUNK // NS


def _deg_call(e3):
    mesh = plsc.VectorSubcoreMesh(core_axis_name="c", subcore_axis_name="s")

    @functools.partial(
        pl.kernel,
        out_type=jax.ShapeDtypeStruct((DEG_R, 128), jnp.float32),
        mesh=mesh,
        scratch_types=[
            pltpu.VMEM((DEG_ROWS_PER_TILE, AGG_CHUNK), jnp.int32),
            pltpu.VMEM((DEG_R, 128), jnp.float32),
            pltpu.VMEM((8, 128), jnp.float32),
            pltpu.VMEM((DEG_R,), jnp.int32),
            pltpu.VMEM_SHARED((DEG_R, 128), jnp.float32),
        ],
        compiler_params=pltpu.CompilerParams(needs_layout_passes=False),
    )
    def call(e3_hbm, dinv_hbm, dstbuf, deg2d, tmp, idx128, deg_sh):
        cid = lax.axis_index("c")
        sid = lax.axis_index("s")

        @pl.when(cid == 0)
        def _():
            zeros16 = jnp.zeros((L,), jnp.float32)
            ones = zeros16 + 1.0

            for i in range(8):
                for j in range(8):
                    tmp[i, pl.ds(j * L, L)] = zeros16
            pltpu.sync_copy(tmp, deg_sh.at[pl.ds(sid * 8, 8)])

            def z(i, _):
                for j in range(8):
                    deg2d[i, pl.ds(j * L, L)] = zeros16
                return 0
            lax.fori_loop(0, DEG_R, z, 0)
            for i in range(8):
                idx128[pl.ds(i * L, L)] = lax.iota(jnp.int32, L) + (i * L)

            pltpu.sync_copy(
                e3_hbm.at[1, pl.ds(sid * DEG_ROWS_PER_TILE,
                                   DEG_ROWS_PER_TILE), :], dstbuf)

            def scat(r, _):
                for u in range(AGG_CHUNK // L):
                    idx = dstbuf[r, pl.ds(u * L, L)]
                    row = lax.shift_right_logical(idx, 7)
                    col = jnp.bitwise_and(idx, 127)
                    plsc.addupdate_scatter(deg2d, [row, col], ones)
                return 0
            lax.fori_loop(0, DEG_ROWS_PER_TILE, scat, 0)

            plsc.subcore_barrier()
            pltpu.sync_copy(deg2d, deg_sh.at[idx128], add=True)
            plsc.subcore_barrier()

            pltpu.sync_copy(deg_sh.at[pl.ds(sid * 8, 8)], tmp)
            for i in range(8):
                for j in range(8):
                    d = tmp[i, pl.ds(j * L, L)] + 1.0
                    tmp[i, pl.ds(j * L, L)] = _newton_rsqrt(d)
            pltpu.sync_copy(tmp, dinv_hbm.at[pl.ds(sid * 8, 8), :])

    return call(e3)


def _agg_call(hs, e3):
    mesh = plsc.VectorSubcoreMesh(core_axis_name="c", subcore_axis_name="s")

    @functools.partial(
        pl.kernel,
        out_type=[jax.ShapeDtypeStruct((NACC, D), jnp.float32),
                  jax.ShapeDtypeStruct((NACC, D), jnp.float32)],
        mesh=mesh,
        scratch_types=[
            pltpu.VMEM((AGG_SEG, AGG_CHUNK), jnp.int32),
            pltpu.VMEM((AGG_SEG, AGG_CHUNK), jnp.int32),
        ] + [pltpu.VMEM((AGG_CHUNK, D), jnp.float32)] * AGG_NBUF + [
            pltpu.VMEM_SHARED((NACC, D), jnp.float32),
        ] + [pltpu.SemaphoreType.DMA] * AGG_NBUF,
        compiler_params=pltpu.CompilerParams(needs_layout_passes=False),
    )
    def call(hs_hbm, e3_hbm, out0_hbm, out1_hbm, sidx, didx, rows0,
             rows1, rows2, rows3, agg_sh, gsem0, gsem1, gsem2, gsem3):
        cid = lax.axis_index("c")
        sid = lax.axis_index("s")
        wid = cid * NS + sid
        rows = [rows0, rows1, rows2, rows3]
        gsems = [gsem0, gsem1, gsem2, gsem3]
        nb = AGG_NBUF

        def zl(i, _):
            for j in range(D // L):
                rows0[i, pl.ds(j * L, L)] = jnp.zeros((L,), jnp.float32)
            return 0
        lax.fori_loop(0, AGG_CHUNK, zl, 0)
        sbase = sid * ACC_PER_TILE
        nz = ACC_PER_TILE // AGG_CHUNK
        for r in range(nz):
            pltpu.sync_copy(rows0, agg_sh.at[pl.ds(sbase + r * AGG_CHUNK,
                                                   AGG_CHUNK)])
        pltpu.sync_copy(rows0.at[pl.ds(0, ACC_PER_TILE - nz * AGG_CHUNK)],
                        agg_sh.at[pl.ds(sbase + nz * AGG_CHUNK,
                                        ACC_PER_TILE - nz * AGG_CHUNK)])
        plsc.subcore_barrier()

        dummy = hs_hbm.at[pl.ds(0, AGG_CHUNK)]
        rowbase = wid * AGG_NCHUNK

        for q in range(AGG_NCHUNK // AGG_SEG):
            qb = rowbase + q * AGG_SEG
            pltpu.sync_copy(e3_hbm.at[0, pl.ds(qb, AGG_SEG), :], sidx)
            pltpu.sync_copy(e3_hbm.at[1, pl.ds(qb, AGG_SEG), :], didx)

            for b in range(nb - 1):
                pltpu.async_copy(hs_hbm.at[sidx.at[b]], rows[b], gsems[b])

            nring = (AGG_SEG - (nb - 1) - 3) // nb

            def ring(g, _):
                k = nb * g
                for b in range(nb):
                    c = k + b
                    fb = (b + nb - 1) % nb
                    pltpu.make_async_copy(dummy, rows[b], gsems[b]).wait()
                    pltpu.async_copy(hs_hbm.at[sidx.at[c + nb - 1]], rows[fb],
                                     gsems[fb])
                    pltpu.sync_copy(rows[b], agg_sh.at[didx.at[c]], add=True)
                return 0
            lax.fori_loop(0, nring, ring, 0)

            for c in range(nring * nb, AGG_SEG):
                b = c % nb
                pltpu.make_async_copy(dummy, rows[b], gsems[b]).wait()
                if c + nb - 1 < AGG_SEG:
                    fb = (b + nb - 1) % nb
                    pltpu.async_copy(hs_hbm.at[sidx.at[c + nb - 1]], rows[fb],
                                     gsems[fb])
                pltpu.sync_copy(rows[b], agg_sh.at[didx.at[c]], add=True)

        plsc.subcore_barrier()

        @pl.when(cid == 0)
        def _():
            pltpu.sync_copy(agg_sh.at[pl.ds(sbase, ACC_PER_TILE)],
                            out0_hbm.at[pl.ds(sbase, ACC_PER_TILE)])

        @pl.when(cid == 1)
        def _():
            pltpu.sync_copy(agg_sh.at[pl.ds(sbase, ACC_PER_TILE)],
                            out1_hbm.at[pl.ds(sbase, ACC_PER_TILE)])

    return call(hs, e3)


_BLKP = 2048
_BLK = 2000
_GRID = 5


def _tcm_body(x_ref, w_ref, o_ref):
    o_ref[...] = jnp.dot(x_ref[...], w_ref[...],
                         preferred_element_type=jnp.float32)


def _tcs_body(u_ref, dinv_ref, o_ref):
    o_ref[...] = u_ref[...] * dinv_ref[...]


def _tc2_body(p0_ref, p1_ref, hs_ref, dinv_ref, b_ref, w_ref, o_ref):
    agg = (p0_ref[...] + p1_ref[...] + hs_ref[...]) * dinv_ref[...]
    x1 = jnp.maximum(agg + b_ref[...], 0.0)
    o_ref[...] = jnp.dot(x1 * dinv_ref[...], w_ref[...],
                         preferred_element_type=jnp.float32)


def _tc3_body(p0_ref, p1_ref, hs_ref, dinv_ref, b_ref, o_ref):
    o_ref[...] = ((p0_ref[...] + p1_ref[...] + hs_ref[...]) * dinv_ref[...]
                  + b_ref[...])


def _row_spec():
    return pl.BlockSpec((_BLK, D), lambda i: (i, 0))


def _rowp_spec():
    return pl.BlockSpec((_BLKP, D), lambda i: (i, 0))


def _dinv_spec():
    return pl.BlockSpec((_BLKP, 1), lambda i: (i, 0))


def _dinv3_spec():
    return pl.BlockSpec((_BLK, 1), lambda i: (i, 0))


def _full_spec():
    return pl.BlockSpec((D, D), lambda i: (0, 0))


def _bias_spec():
    return pl.BlockSpec((1, D), lambda i: (0, 0))


def _tcm(x, w):
    return pl.pallas_call(
        _tcm_body,
        grid=(_GRID,),
        in_specs=[_rowp_spec(), _full_spec()],
        out_specs=_rowp_spec(),
        out_shape=jax.ShapeDtypeStruct((NPAD, D), jnp.float32),
    )(x, w)


def _tcs(u, dinv_col):
    return pl.pallas_call(
        _tcs_body,
        grid=(_GRID,),
        in_specs=[_rowp_spec(), _dinv_spec()],
        out_specs=_rowp_spec(),
        out_shape=jax.ShapeDtypeStruct((NPAD, D), jnp.float32),
    )(u, dinv_col)


def _tc2(p0, p1, hs, dinv_bc, b, w):
    return pl.pallas_call(
        _tc2_body,
        grid=(_GRID,),
        in_specs=[_rowp_spec(), _rowp_spec(), _rowp_spec(), _dinv_spec(),
                  _bias_spec(), _full_spec()],
        out_specs=_rowp_spec(),
        out_shape=jax.ShapeDtypeStruct((NPAD, D), jnp.float32),
    )(p0, p1, hs, dinv_bc, b, w)


def _tc3(p0, p1, hs, dinv_bc, b):
    return pl.pallas_call(
        _tc3_body,
        grid=(_GRID,),
        in_specs=[_row_spec(), _row_spec(), _row_spec(), _dinv3_spec(),
                  _bias_spec()],
        out_specs=_row_spec(),
        out_shape=jax.ShapeDtypeStruct((N_GENES, D), jnp.float32),
    )(p0, p1, hs, dinv_bc, b)


def kernel(gene_ind_vec, edge_index, gene_embedding, W1, b1, W2, b2):
    pad3 = jnp.asarray(np.stack([_SRC_PAD, _DST_PAD]))
    e3 = jnp.concatenate(
        [edge_index.reshape(2, -1, AGG_CHUNK), pad3], axis=1)

    u1 = _tcm(gene_embedding, W1)
    dinv2d = _deg_call(e3)
    dinv_col = dinv2d.reshape(-1)[:NPAD, None]

    hs1 = _tcs(u1, dinv_col)
    p0, p1 = _agg_call(hs1, e3)
    hs2 = _tc2(p0, p1, hs1, dinv_col, b1.reshape(1, D), W2)
    q0, q1 = _agg_call(hs2, e3)
    out = _tc3(q0, q1, hs2, dinv_col, b2.reshape(1, D))
    return out


# speedup vs baseline: 1.0603x; 1.0010x over previous
import functools

import jax
import jax.numpy as jnp
import numpy as np
from jax import lax
from jax.experimental import pallas as pl
from jax.experimental.pallas import tpu as pltpu
from jax.experimental.pallas import tpu_sc as plsc

N_GENES = 10000
D = 128
N_EDGES = 320000

NC = 2
NS = 16
L = 16

NPAD = 10240
AGG_CHUNK = 80
AGG_NCHUNK = 128
AGG_SEG = 32
AGG_NBUF = 4
EPAD = NC * NS * AGG_NCHUNK * AGG_CHUNK
NACC = 10112
ACC_PER_TILE = NACC // NS

_N_EPAD = EPAD - N_EDGES
_SRC_PAD = np.asarray((np.arange(_N_EPAD) * 131) % N_GENES,
                      np.int32).reshape(-1, AGG_CHUNK)
_DST_PAD = np.asarray(N_GENES + np.arange(_N_EPAD) % (NACC - N_GENES),
                      np.int32).reshape(-1, AGG_CHUNK)


def _newton_rsqrt(x):
    i = plsc.bitcast(x, jnp.int32)
    y = plsc.bitcast(jnp.int32(0x5F3759DF) - (i >> 1), jnp.float32)
    for _ in range(3):
        y = y * (1.5 - 0.5 * x * y * y)
    return y


DEG_R = 128
DEG_ROWS_PER_TILE = EPAD // AGG_CHUNK // NS


def _deg_call(e3):
    mesh = plsc.VectorSubcoreMesh(core_axis_name="c", subcore_axis_name="s")

    @functools.partial(
        pl.kernel,
        out_type=jax.ShapeDtypeStruct((DEG_R, 128), jnp.float32),
        mesh=mesh,
        scratch_types=[
            pltpu.VMEM((DEG_ROWS_PER_TILE, AGG_CHUNK), jnp.int32),
            pltpu.VMEM((DEG_R, 128), jnp.float32),
            pltpu.VMEM((8, 128), jnp.float32),
            pltpu.VMEM((DEG_R,), jnp.int32),
            pltpu.VMEM_SHARED((DEG_R, 128), jnp.float32),
        ],
        compiler_params=pltpu.CompilerParams(needs_layout_passes=False),
    )
    def call(e3_hbm, dinv_hbm, dstbuf, deg2d, tmp, idx128, deg_sh):
        cid = lax.axis_index("c")
        sid = lax.axis_index("s")

        @pl.when(cid == 0)
        def _():
            zeros16 = jnp.zeros((L,), jnp.float32)
            ones = zeros16 + 1.0

            for i in range(8):
                for j in range(8):
                    tmp[i, pl.ds(j * L, L)] = zeros16
            pltpu.sync_copy(tmp, deg_sh.at[pl.ds(sid * 8, 8)])

            def z(i, _):
                for j in range(8):
                    deg2d[i, pl.ds(j * L, L)] = zeros16
                return 0
            lax.fori_loop(0, DEG_R, z, 0)
            for i in range(8):
                idx128[pl.ds(i * L, L)] = lax.iota(jnp.int32, L) + (i * L)

            pltpu.sync_copy(
                e3_hbm.at[1, pl.ds(sid * DEG_ROWS_PER_TILE,
                                   DEG_ROWS_PER_TILE), :], dstbuf)

            def scat(r, _):
                for u in range(AGG_CHUNK // L):
                    idx = dstbuf[r, pl.ds(u * L, L)]
                    row = lax.shift_right_logical(idx, 7)
                    col = jnp.bitwise_and(idx, 127)
                    plsc.addupdate_scatter(deg2d, [row, col], ones)
                return 0
            lax.fori_loop(0, DEG_ROWS_PER_TILE, scat, 0)

            plsc.subcore_barrier()
            pltpu.sync_copy(deg2d, deg_sh.at[idx128], add=True)
            plsc.subcore_barrier()

            pltpu.sync_copy(deg_sh.at[pl.ds(sid * 8, 8)], tmp)
            for i in range(8):
                for j in range(8):
                    d = tmp[i, pl.ds(j * L, L)] + 1.0
                    tmp[i, pl.ds(j * L, L)] = _newton_rsqrt(d)
            pltpu.sync_copy(tmp, dinv_hbm.at[pl.ds(sid * 8, 8), :])

    return call(e3)


def _agg_call(hs, e3):
    mesh = plsc.VectorSubcoreMesh(core_axis_name="c", subcore_axis_name="s")

    @functools.partial(
        pl.kernel,
        out_type=[jax.ShapeDtypeStruct((NACC, D), jnp.float32),
                  jax.ShapeDtypeStruct((NACC, D), jnp.float32)],
        mesh=mesh,
        scratch_types=[
            pltpu.VMEM((AGG_SEG, AGG_CHUNK), jnp.int32),
            pltpu.VMEM((AGG_SEG, AGG_CHUNK), jnp.int32),
        ] + [pltpu.VMEM((AGG_CHUNK, D), jnp.float32)] * AGG_NBUF + [
            pltpu.VMEM_SHARED((NACC, D), jnp.float32),
        ] + [pltpu.SemaphoreType.DMA] * AGG_NBUF,
        compiler_params=pltpu.CompilerParams(needs_layout_passes=False),
    )
    def call(hs_hbm, e3_hbm, out0_hbm, out1_hbm, sidx, didx, rows0,
             rows1, rows2, rows3, agg_sh, gsem0, gsem1, gsem2, gsem3):
        cid = lax.axis_index("c")
        sid = lax.axis_index("s")
        wid = cid * NS + sid
        rows = [rows0, rows1, rows2, rows3]
        gsems = [gsem0, gsem1, gsem2, gsem3]
        nb = AGG_NBUF

        def zl(i, _):
            for j in range(D // L):
                rows0[i, pl.ds(j * L, L)] = jnp.zeros((L,), jnp.float32)
            return 0
        lax.fori_loop(0, AGG_CHUNK, zl, 0)
        sbase = sid * ACC_PER_TILE
        nz = ACC_PER_TILE // AGG_CHUNK
        for r in range(nz):
            pltpu.sync_copy(rows0, agg_sh.at[pl.ds(sbase + r * AGG_CHUNK,
                                                   AGG_CHUNK)])
        pltpu.sync_copy(rows0.at[pl.ds(0, ACC_PER_TILE - nz * AGG_CHUNK)],
                        agg_sh.at[pl.ds(sbase + nz * AGG_CHUNK,
                                        ACC_PER_TILE - nz * AGG_CHUNK)])
        plsc.subcore_barrier()

        dummy = hs_hbm.at[pl.ds(0, AGG_CHUNK)]
        rowbase = wid * AGG_NCHUNK

        for q in range(AGG_NCHUNK // AGG_SEG):
            qb = rowbase + q * AGG_SEG
            pltpu.sync_copy(e3_hbm.at[0, pl.ds(qb, AGG_SEG), :], sidx)
            pltpu.sync_copy(e3_hbm.at[1, pl.ds(qb, AGG_SEG), :], didx)

            for b in range(nb - 1):
                pltpu.async_copy(hs_hbm.at[sidx.at[b]], rows[b], gsems[b])

            nring = (AGG_SEG - (nb - 1) - 3) // nb

            def ring(g, _):
                k = nb * g
                for b in range(nb):
                    c = k + b
                    fb = (b + nb - 1) % nb
                    pltpu.make_async_copy(dummy, rows[b], gsems[b]).wait()
                    pltpu.async_copy(hs_hbm.at[sidx.at[c + nb - 1]], rows[fb],
                                     gsems[fb])
                    pltpu.sync_copy(rows[b], agg_sh.at[didx.at[c]], add=True)
                return 0
            lax.fori_loop(0, nring, ring, 0)

            for c in range(nring * nb, AGG_SEG):
                b = c % nb
                pltpu.make_async_copy(dummy, rows[b], gsems[b]).wait()
                if c + nb - 1 < AGG_SEG:
                    fb = (b + nb - 1) % nb
                    pltpu.async_copy(hs_hbm.at[sidx.at[c + nb - 1]], rows[fb],
                                     gsems[fb])
                pltpu.sync_copy(rows[b], agg_sh.at[didx.at[c]], add=True)

        plsc.subcore_barrier()

        @pl.when(cid == 0)
        def _():
            pltpu.sync_copy(agg_sh.at[pl.ds(sbase, ACC_PER_TILE)],
                            out0_hbm.at[pl.ds(sbase, ACC_PER_TILE)])

        @pl.when(cid == 1)
        def _():
            pltpu.sync_copy(agg_sh.at[pl.ds(sbase, ACC_PER_TILE)],
                            out1_hbm.at[pl.ds(sbase, ACC_PER_TILE)])

    return call(hs, e3)


_BLKP = 2048
_BLK = 2000
_GRID = 5


def _tcm_body(x_ref, w_ref, o_ref):
    o_ref[...] = jnp.dot(x_ref[...], w_ref[...],
                         preferred_element_type=jnp.float32)


def _tcs_body(u_ref, dinv_ref, o_ref):
    o_ref[...] = u_ref[...] * dinv_ref[...]


def _tc2_body(p0_ref, p1_ref, hs_ref, dinv_ref, b_ref, w_ref, o_ref):
    agg = (p0_ref[...] + p1_ref[...] + hs_ref[...]) * dinv_ref[...]
    x1 = jnp.maximum(agg + b_ref[...], 0.0)
    o_ref[...] = jnp.dot(x1 * dinv_ref[...], w_ref[...],
                         preferred_element_type=jnp.float32)


def _tc3_body(p0_ref, p1_ref, hs_ref, dinv_ref, b_ref, o_ref):
    o_ref[...] = ((p0_ref[...] + p1_ref[...] + hs_ref[...]) * dinv_ref[...]
                  + b_ref[...])


def _row_spec():
    return pl.BlockSpec((_BLK, D), lambda i: (i, 0))


def _rowp_spec():
    return pl.BlockSpec((_BLKP, D), lambda i: (i, 0))


def _dinv_spec():
    return pl.BlockSpec((_BLKP, 1), lambda i: (i, 0))


def _dinv3_spec():
    return pl.BlockSpec((_BLK, 1), lambda i: (i, 0))


def _full_spec():
    return pl.BlockSpec((D, D), lambda i: (0, 0))


def _bias_spec():
    return pl.BlockSpec((1, D), lambda i: (0, 0))


def _tcm(x, w):
    return pl.pallas_call(
        _tcm_body,
        grid=(_GRID,),
        in_specs=[_rowp_spec(), _full_spec()],
        out_specs=_rowp_spec(),
        out_shape=jax.ShapeDtypeStruct((NPAD, D), jnp.float32),
    )(x, w)


def _tcs(u, dinv_col):
    return pl.pallas_call(
        _tcs_body,
        grid=(_GRID,),
        in_specs=[_rowp_spec(), _dinv_spec()],
        out_specs=_rowp_spec(),
        out_shape=jax.ShapeDtypeStruct((NPAD, D), jnp.float32),
    )(u, dinv_col)


def _tc2(p0, p1, hs, dinv_bc, b, w):
    return pl.pallas_call(
        _tc2_body,
        grid=(_GRID,),
        in_specs=[_rowp_spec(), _rowp_spec(), _rowp_spec(), _dinv_spec(),
                  _bias_spec(), _full_spec()],
        out_specs=_rowp_spec(),
        out_shape=jax.ShapeDtypeStruct((NPAD, D), jnp.float32),
    )(p0, p1, hs, dinv_bc, b, w)


def _tc3(p0, p1, hs, dinv_bc, b):
    return pl.pallas_call(
        _tc3_body,
        grid=(_GRID,),
        in_specs=[_row_spec(), _row_spec(), _row_spec(), _dinv3_spec(),
                  _bias_spec()],
        out_specs=_row_spec(),
        out_shape=jax.ShapeDtypeStruct((N_GENES, D), jnp.float32),
    )(p0, p1, hs, dinv_bc, b)


def kernel(gene_ind_vec, edge_index, gene_embedding, W1, b1, W2, b2):
    pad3 = jnp.asarray(np.stack([_SRC_PAD, _DST_PAD]))
    e3 = jnp.concatenate(
        [edge_index.reshape(2, -1, AGG_CHUNK), pad3], axis=1)

    u1 = _tcm(gene_embedding, W1)
    dinv2d = _deg_call(e3)
    dinv_col = dinv2d.reshape(-1)[:NPAD, None]

    hs1 = _tcs(u1, dinv_col)
    p0, p1 = _agg_call(hs1, e3)
    hs2 = _tc2(p0, p1, hs1, dinv_col, b1.reshape(1, D), W2)
    q0, q1 = _agg_call(hs2, e3)
    out = _tc3(q0, q1, hs2, dinv_col, b2.reshape(1, D))
    return out
